# baseline (speedup 1.0000x reference)
import functools

import jax
import jax.numpy as jnp
from jax import lax
from jax.experimental import pallas as pl
from jax.experimental.pallas import tpu as pltpu

BETA_ = 0.25
TM = 1024
CK = 1024


def _e2_kernel(emb_ref, e2_ref):
    e = emb_ref[...]
    e2_ref[...] = jnp.sum(e * e, axis=1)[None, :]


def _vq_tc_kernel(zf_ref, emb_ref, e2_ref, idx_ref, loss_ref):
    k, d = emb_ref.shape
    nc = k // CK

    z = zf_ref[...]
    x2 = jnp.sum(z * z, axis=1, keepdims=True)

    def dotc(c):
        e_c = emb_ref[c * CK:(c + 1) * CK, :]
        return lax.dot_general(z, e_c, (((1,), (1,)), ((), ())),
                               preferred_element_type=jnp.float32)

    run_m = None
    run_i = None
    d_prev = dotc(0)
    for c in range(nc):
        d_next = dotc(c + 1) if c + 1 < nc else None
        d2 = x2 - 2.0 * d_prev + e2_ref[:, c * CK:(c + 1) * CK]
        dist = jnp.sqrt(jnp.maximum(d2, 0.0))
        mck = jnp.min(dist, axis=1, keepdims=True)
        col = lax.broadcasted_iota(jnp.int32, dist.shape, 1) + (c * CK)
        ic = jnp.min(jnp.where(dist == mck, col, k), axis=1)
        mc = mck[:, 0]
        if c == 0:
            run_m, run_i = mc, ic
        else:
            better = mc < run_m
            run_i = jnp.where(better, ic, run_i)
            run_m = jnp.where(better, mc, run_m)
        d_prev = d_next

    idx_ref[...] = run_i
    loss_ref[...] = jnp.sum(run_m * run_m).reshape(1, 1, 1)


@jax.jit
def _vq_core(z_flat, embedding_weight):
    n, d = z_flat.shape
    k = embedding_weight.shape[0]
    e2 = pl.pallas_call(
        _e2_kernel,
        out_shape=jax.ShapeDtypeStruct((1, k), jnp.float32),
    )(embedding_weight)
    grid = (n // TM,)
    idx, loss_parts = pl.pallas_call(
        _vq_tc_kernel,
        grid=grid,
        in_specs=[
            pl.BlockSpec((TM, d), lambda i: (i, 0)),
            pl.BlockSpec((k, d), lambda i: (0, 0)),
            pl.BlockSpec((1, k), lambda i: (0, 0)),
        ],
        out_specs=[
            pl.BlockSpec((TM,), lambda i: (i,)),
            pl.BlockSpec((1, 1, 1), lambda i: (i, 0, 0)),
        ],
        out_shape=[
            jax.ShapeDtypeStruct((n,), jnp.int32),
            jax.ShapeDtypeStruct((n // TM, 1, 1), jnp.float32),
        ],
        compiler_params=pltpu.CompilerParams(
            dimension_semantics=("parallel",)),
    )(z_flat, embedding_weight, e2)
    return idx, jnp.sum(loss_parts)


def kernel(z, embedding_weight):
    zp = jnp.transpose(z, (0, 2, 3, 1))
    z_shape = zp.shape
    z_flat = zp.reshape(-1, embedding_weight.shape[1])
    idx, loss_sum = _vq_core(z_flat, embedding_weight)
    z_q = jnp.take(embedding_weight, idx, axis=0).reshape(z_shape)
    n = z_flat.shape[0] * z_flat.shape[1]
    loss = loss_sum / n + BETA_ * (loss_sum / n)
    z_q_st = zp + lax.stop_gradient(z_q - zp)
    z_q_out = jnp.transpose(z_q_st, (0, 3, 1, 2))
    return (z_q_out, loss, (idx, z_flat))


# speedup vs baseline: 1.2296x; 1.2296x over previous
import jax
import jax.numpy as jnp
from jax import lax
from jax.experimental import pallas as pl
from jax.experimental.pallas import tpu as pltpu

BETA_ = 0.25
TM = 256
CK = 1024


def _bits_add(x, delta):
    return lax.bitcast_convert_type(
        lax.bitcast_convert_type(x, jnp.int32) + delta, jnp.float32)


def _vq_tc_kernel(zf_ref, emb_ref, idx_ref, loss_ref, e2_ref, d2s_ref):
    i = pl.program_id(0)
    k, d = emb_ref.shape
    nc = k // CK

    @pl.when(i == 0)
    def _():
        e = emb_ref[...]
        e2_ref[...] = jnp.sum(e * e, axis=1)[None, :]
        loss_ref[...] = jnp.zeros((1, 1), jnp.float32)

    z = zf_ref[...]
    x2 = jnp.sum(z * z, axis=1, keepdims=True)

    def dotc(c):
        e_c = emb_ref[c * CK:(c + 1) * CK, :]
        return lax.dot_general(z, e_c, (((1,), (1,)), ((), ())),
                               preferred_element_type=jnp.float32)

    m2 = None
    d_prev = dotc(0)
    for c in range(nc):
        d_next = dotc(c + 1) if c + 1 < nc else None
        d2 = x2 - 2.0 * d_prev + e2_ref[:, c * CK:(c + 1) * CK]
        d2s_ref[:, c * CK:(c + 1) * CK] = d2
        m2c = jnp.min(d2, axis=1, keepdims=True)
        m2 = m2c if c == 0 else jnp.minimum(m2, m2c)
        d_prev = d_next

    s = jnp.sqrt(jnp.maximum(m2, 0.0))
    t = s * _bits_add(s, 1)
    for _ in range(3):
        bad = jnp.sqrt(jnp.maximum(t, 0.0)) > s
        t = jnp.where(bad, _bits_add(t, -1), t)
    for _ in range(3):
        tu = _bits_add(t, 1)
        ok = jnp.sqrt(jnp.maximum(tu, 0.0)) <= s
        t = jnp.where(ok, tu, t)

    d2all = d2s_ref[...]
    col = lax.broadcasted_iota(jnp.int32, (TM, k), 1)
    idx = jnp.min(jnp.where(d2all <= t, col, k), axis=1)
    idx_ref[...] = idx
    loss_ref[...] += jnp.sum(m2[:, 0]).reshape(1, 1)


@jax.jit
def _vq_core(z_flat, embedding_weight):
    n, d = z_flat.shape
    k = embedding_weight.shape[0]
    grid = (n // TM,)
    idx, loss_sum = pl.pallas_call(
        _vq_tc_kernel,
        grid=grid,
        in_specs=[
            pl.BlockSpec((TM, d), lambda i: (i, 0)),
            pl.BlockSpec((k, d), lambda i: (0, 0)),
        ],
        out_specs=[
            pl.BlockSpec((TM,), lambda i: (i,)),
            pl.BlockSpec((1, 1), lambda i: (0, 0)),
        ],
        out_shape=[
            jax.ShapeDtypeStruct((n,), jnp.int32),
            jax.ShapeDtypeStruct((1, 1), jnp.float32),
        ],
        scratch_shapes=[
            pltpu.VMEM((1, k), jnp.float32),
            pltpu.VMEM((TM, k), jnp.float32),
        ],
    )(z_flat, embedding_weight)
    return idx, loss_sum[0, 0]


def kernel(z, embedding_weight):
    zp = jnp.transpose(z, (0, 2, 3, 1))
    z_shape = zp.shape
    z_flat = zp.reshape(-1, embedding_weight.shape[1])
    idx, loss_sum = _vq_core(z_flat, embedding_weight)
    z_q = jnp.take(embedding_weight, idx, axis=0).reshape(z_shape)
    n = z_flat.shape[0] * z_flat.shape[1]
    loss = loss_sum / n + BETA_ * (loss_sum / n)
    z_q_st = zp + lax.stop_gradient(z_q - zp)
    z_q_out = jnp.transpose(z_q_st, (0, 3, 1, 2))
    return (z_q_out, loss, (idx, z_flat))
